# baseline (speedup 1.0000x reference)
import functools

import jax
import jax.numpy as jnp
from jax import lax
from jax.experimental import pallas as pl
from jax.experimental.pallas import tpu as pltpu
from jax.experimental.pallas import tpu_sc as plsc

AUDIO_DIM = 128
TEXT_DIM = 768
EPS = 1e-06

BATCH = 4096
HIST = 20
HIST_PAD = 24

NC = 2
NS = 16
NW = NC * NS
CHUNK = 128
GROUP = 4
N_ROWS = BATCH * HIST_PAD
B_PER_W = N_ROWS // NW
N_CHUNKS = B_PER_W // CHUNK
N_GROUPS = N_CHUNKS // GROUP

BB = 64


def _sc_gather(table, idx3):
    mesh = plsc.VectorSubcoreMesh(core_axis_name="c", subcore_axis_name="s")

    @functools.partial(
        pl.kernel,
        out_type=jax.ShapeDtypeStruct((N_ROWS, AUDIO_DIM), jnp.float32),
        mesh=mesh,
        scratch_types=[
            pltpu.VMEM((N_CHUNKS, CHUNK), jnp.int32),
            pltpu.VMEM((GROUP * CHUNK, AUDIO_DIM), jnp.float32),
            pltpu.SemaphoreType.DMA,
        ],
    )
    def k(table_hbm, idx_hbm, out_hbm, idx_v, rows_v, sem):
        wid = lax.axis_index("s") * NC + lax.axis_index("c")
        base = wid * B_PER_W
        pltpu.sync_copy(idx_hbm.at[wid], idx_v)
        for g in range(N_GROUPS):
            copies = [
                pltpu.async_copy(
                    table_hbm.at[idx_v.at[g * GROUP + b]],
                    rows_v.at[pl.ds(b * CHUNK, CHUNK)],
                    sem,
                )
                for b in range(GROUP)
            ]
            for cp in copies:
                cp.wait()
            pltpu.sync_copy(
                rows_v, out_hbm.at[pl.ds(base + g * GROUP * CHUNK, GROUP * CHUNK)]
            )

    return k(table, idx3)


def _tc_dense(x, scale, w):
    grid = (BATCH // BB,)

    def body(x_ref, s_ref, w_ref, o_ref):
        xv = x_ref[...]
        var = jnp.mean(xv * xv, axis=-1, keepdims=True)
        xn = xv * lax.rsqrt(var + EPS) * s_ref[...]
        p = jnp.dot(xn, w_ref[...], preferred_element_type=jnp.float32)
        var2 = jnp.mean(p * p, axis=-1, keepdims=True)
        r = p * lax.rsqrt(var2 + EPS)
        o_ref[...] = r.reshape(BB, HIST_PAD, TEXT_DIM)

    return pl.pallas_call(
        body,
        grid=grid,
        in_specs=[
            pl.BlockSpec((BB * HIST_PAD, AUDIO_DIM), lambda i: (i, 0)),
            pl.BlockSpec((1, AUDIO_DIM), lambda i: (0, 0)),
            pl.BlockSpec((AUDIO_DIM, TEXT_DIM), lambda i: (0, 0)),
        ],
        out_specs=pl.BlockSpec((BB, HIST_PAD, TEXT_DIM), lambda i: (i, 0, 0)),
        out_shape=jax.ShapeDtypeStruct((BATCH, HIST_PAD, TEXT_DIM), jnp.float32),
    )(x, scale, w)


def kernel(input_ids, table, norm_scale, proj_w):
    batch, hist = input_ids.shape
    ids = input_ids.astype(jnp.int32)
    pads = jnp.arange(batch * (HIST_PAD - hist), dtype=jnp.int32).reshape(
        batch, HIST_PAD - hist
    )
    ids_pad = jnp.concatenate([ids, pads], axis=1)
    idx3 = ids_pad.reshape(NW, N_CHUNKS, CHUNK)
    gathered = _sc_gather(table, idx3)
    out24 = _tc_dense(gathered, norm_scale.reshape(1, AUDIO_DIM), proj_w.T)
    return out24[:, :HIST, :]


# speedup vs baseline: 2.4207x; 2.4207x over previous
import functools

import jax
import jax.numpy as jnp
from jax import lax
from jax.experimental import pallas as pl
from jax.experimental.pallas import tpu as pltpu
from jax.experimental.pallas import tpu_sc as plsc

AUDIO_DIM = 128
TEXT_DIM = 768
EPS = 1e-06

BATCH = 4096
HIST = 20

NC = 2
NS = 16
NW = NC * NS
CHUNK = 128
GROUP = 4
N_TOKENS = BATCH * HIST
B_PER_W = N_TOKENS // NW
N_CHUNKS = B_PER_W // CHUNK
N_GROUPS = N_CHUNKS // GROUP

ROWS = 2048


def _sc_gather(table, idx3):
    mesh = plsc.VectorSubcoreMesh(core_axis_name="c", subcore_axis_name="s")

    @functools.partial(
        pl.kernel,
        out_type=jax.ShapeDtypeStruct((N_TOKENS, AUDIO_DIM), jnp.float32),
        mesh=mesh,
        scratch_types=[
            pltpu.VMEM((N_CHUNKS, CHUNK), jnp.int32),
            pltpu.VMEM((GROUP * CHUNK, AUDIO_DIM), jnp.float32),
            pltpu.SemaphoreType.DMA,
        ],
    )
    def k(table_hbm, idx_hbm, out_hbm, idx_v, rows_v, sem):
        wid = lax.axis_index("s") * NC + lax.axis_index("c")
        base = wid * B_PER_W
        pltpu.sync_copy(idx_hbm.at[wid], idx_v)
        for g in range(N_GROUPS):
            copies = [
                pltpu.async_copy(
                    table_hbm.at[idx_v.at[g * GROUP + b]],
                    rows_v.at[pl.ds(b * CHUNK, CHUNK)],
                    sem,
                )
                for b in range(GROUP)
            ]
            for cp in copies:
                cp.wait()
            pltpu.sync_copy(
                rows_v, out_hbm.at[pl.ds(base + g * GROUP * CHUNK, GROUP * CHUNK)]
            )

    return k(table, idx3)


def _tc_dense(x, scale, w):
    grid = (N_TOKENS // ROWS,)

    def body(x_ref, s_ref, w_ref, o_ref):
        xv = x_ref[...]
        var = jnp.mean(xv * xv, axis=-1, keepdims=True)
        xn = xv * lax.rsqrt(var + EPS) * s_ref[...]
        p = jnp.dot(xn, w_ref[...], preferred_element_type=jnp.float32)
        var2 = jnp.mean(p * p, axis=-1, keepdims=True)
        o_ref[...] = p * lax.rsqrt(var2 + EPS)

    return pl.pallas_call(
        body,
        grid=grid,
        in_specs=[
            pl.BlockSpec((ROWS, AUDIO_DIM), lambda i: (i, 0)),
            pl.BlockSpec((1, AUDIO_DIM), lambda i: (0, 0)),
            pl.BlockSpec((AUDIO_DIM, TEXT_DIM), lambda i: (0, 0)),
        ],
        out_specs=pl.BlockSpec((ROWS, TEXT_DIM), lambda i: (i, 0)),
        out_shape=jax.ShapeDtypeStruct((N_TOKENS, TEXT_DIM), jnp.float32),
    )(x, scale, w)


def kernel(input_ids, table, norm_scale, proj_w):
    batch, hist = input_ids.shape
    ids_t = input_ids.T.astype(jnp.int32)
    idx3 = ids_t.reshape(NW, N_CHUNKS, CHUNK)
    gathered = _sc_gather(table, idx3)
    out = _tc_dense(gathered, norm_scale.reshape(1, AUDIO_DIM), proj_w.T)
    return out.reshape(hist, batch, TEXT_DIM).transpose(1, 0, 2)


# speedup vs baseline: 2.4875x; 1.0276x over previous
import functools

import jax
import jax.numpy as jnp
from jax import lax
from jax.experimental import pallas as pl
from jax.experimental.pallas import tpu as pltpu
from jax.experimental.pallas import tpu_sc as plsc

AUDIO_DIM = 128
TEXT_DIM = 768
EPS = 1e-06

BATCH = 4096
HIST = 20

NC = 2
NS = 16
NW = NC * NS
CHUNK = 128
GROUP = 4
N_TOKENS = BATCH * HIST
B_PER_W = N_TOKENS // NW
N_CHUNKS = B_PER_W // CHUNK
N_GROUPS = N_CHUNKS // GROUP

ROWS = 4096


def _sc_gather(table, idx3):
    mesh = plsc.VectorSubcoreMesh(core_axis_name="c", subcore_axis_name="s")

    @functools.partial(
        pl.kernel,
        out_type=jax.ShapeDtypeStruct((N_TOKENS, AUDIO_DIM), jnp.float32),
        mesh=mesh,
        scratch_types=[
            pltpu.VMEM((N_CHUNKS, CHUNK), jnp.int32),
            pltpu.VMEM((GROUP * CHUNK, AUDIO_DIM), jnp.float32),
            pltpu.SemaphoreType.DMA,
        ],
    )
    def k(table_hbm, idx_hbm, out_hbm, idx_v, rows_v, sem):
        wid = lax.axis_index("s") * NC + lax.axis_index("c")
        base = wid * B_PER_W
        pltpu.sync_copy(idx_hbm.at[wid], idx_v)
        for g in range(N_GROUPS):
            copies = [
                pltpu.async_copy(
                    table_hbm.at[idx_v.at[g * GROUP + b]],
                    rows_v.at[pl.ds(b * CHUNK, CHUNK)],
                    sem,
                )
                for b in range(GROUP)
            ]
            for cp in copies:
                cp.wait()
            pltpu.sync_copy(
                rows_v, out_hbm.at[pl.ds(base + g * GROUP * CHUNK, GROUP * CHUNK)]
            )

    return k(table, idx3)


def _tc_dense(x, scale, w):
    grid = (N_TOKENS // ROWS,)

    def body(x_ref, s_ref, w_ref, o_ref):
        xv = x_ref[...]
        var = jnp.mean(xv * xv, axis=-1, keepdims=True)
        xn = xv * lax.rsqrt(var + EPS) * s_ref[...]
        p = jnp.dot(xn, w_ref[...], preferred_element_type=jnp.float32)
        var2 = jnp.mean(p * p, axis=-1, keepdims=True)
        o_ref[...] = p * lax.rsqrt(var2 + EPS)

    return pl.pallas_call(
        body,
        grid=grid,
        in_specs=[
            pl.BlockSpec((ROWS, AUDIO_DIM), lambda i: (i, 0)),
            pl.BlockSpec((1, AUDIO_DIM), lambda i: (0, 0)),
            pl.BlockSpec((AUDIO_DIM, TEXT_DIM), lambda i: (0, 0)),
        ],
        out_specs=pl.BlockSpec((ROWS, TEXT_DIM), lambda i: (i, 0)),
        out_shape=jax.ShapeDtypeStruct((N_TOKENS, TEXT_DIM), jnp.float32),
    )(x, scale, w)


def kernel(input_ids, table, norm_scale, proj_w):
    batch, hist = input_ids.shape
    ids_t = input_ids.T.astype(jnp.int32)
    idx3 = ids_t.reshape(NW, N_CHUNKS, CHUNK)
    gathered = _sc_gather(table, idx3)
    out = _tc_dense(gathered, norm_scale.reshape(1, AUDIO_DIM), proj_w.T)
    return out.reshape(hist, batch, TEXT_DIM).transpose(1, 0, 2)
